# baseline (speedup 1.0000x reference)
import jax
import jax.numpy as jnp
from jax.experimental import pallas as pl

_BLOCK = 1000


def _fused_body(x_ref, w_ref, g_ref, o_ref):
    xb = x_ref[...]
    var = jnp.mean(xb * xb, axis=-1, keepdims=True)
    normed = xb * jax.lax.rsqrt(var + 1e-6) * g_ref[...]
    o_ref[...] = xb + jax.lax.dot_general(
        normed, w_ref[...],
        dimension_numbers=(((1,), (1,)), ((), ())),
        preferred_element_type=jnp.float32,
    )


def kernel(x, edge_index, W, scoring_src, scoring_tgt, ln_weight):
    n, d = x.shape
    grid = (n // _BLOCK,) if n % _BLOCK == 0 else (pl.cdiv(n, _BLOCK),)
    out = pl.pallas_call(
        _fused_body,
        grid=grid,
        in_specs=[
            pl.BlockSpec((_BLOCK, d), lambda i: (i, 0)),
            pl.BlockSpec((d, d), lambda i: (0, 0)),
            pl.BlockSpec((1, d), lambda i: (0, 0)),
        ],
        out_specs=pl.BlockSpec((_BLOCK, d), lambda i: (i, 0)),
        out_shape=jax.ShapeDtypeStruct((n, d), x.dtype),
    )(x, W, ln_weight.reshape(1, d))
    return (out, edge_index)


# speedup vs baseline: 1.2505x; 1.2505x over previous
import jax
import jax.numpy as jnp
from jax.experimental import pallas as pl

_BLOCK = 5000


def _fused_body(x_ref, w_ref, g_ref, o_ref):
    xb = x_ref[...]
    var = jnp.mean(xb * xb, axis=-1, keepdims=True)
    normed = xb * jax.lax.rsqrt(var + 1e-6) * g_ref[...]
    o_ref[...] = xb + jax.lax.dot_general(
        normed, w_ref[...],
        dimension_numbers=(((1,), (1,)), ((), ())),
        preferred_element_type=jnp.float32,
    )


def kernel(x, edge_index, W, scoring_src, scoring_tgt, ln_weight):
    n, d = x.shape
    grid = (n // _BLOCK,) if n % _BLOCK == 0 else (pl.cdiv(n, _BLOCK),)
    out = pl.pallas_call(
        _fused_body,
        grid=grid,
        in_specs=[
            pl.BlockSpec((_BLOCK, d), lambda i: (i, 0)),
            pl.BlockSpec((d, d), lambda i: (0, 0)),
            pl.BlockSpec((1, d), lambda i: (0, 0)),
        ],
        out_specs=pl.BlockSpec((_BLOCK, d), lambda i: (i, 0)),
        out_shape=jax.ShapeDtypeStruct((n, d), x.dtype),
    )(x, W, ln_weight.reshape(1, d))
    return (out, edge_index)
